# Initial kernel scaffold; baseline (speedup 1.0000x reference)
import jax
import jax.numpy as jnp
from jax.experimental import pallas as pl


def kernel(x, edge_index):
    raise NotImplementedError("write your pallas kernel here")


# speedup vs baseline: 1.4702x; 1.4702x over previous
import jax
import jax.numpy as jnp
import numpy as np
from jax import lax
from jax.experimental import pallas as pl
from jax.experimental.pallas import tpu as pltpu
from jax.experimental.pallas import tpu_sc as plsc

N_NODES = 10000
N_EDGES = 160000
D_FEAT = 256

NUM_CORES = 2
NUM_SUBCORES = 16
NUM_WORKERS = NUM_CORES * NUM_SUBCORES

EDGES_PER_WORKER = N_EDGES // NUM_WORKERS
BLOCK_E = 200
LANES = 16
NUM_GROUPS = (BLOCK_E + LANES - 1) // LANES
BLOCK_PAD = NUM_GROUPS * LANES
NUM_BLOCKS = EDGES_PER_WORKER // BLOCK_E
D_CHUNKS = D_FEAT // LANES


def _permute(v, perm):
    return lax.gather(
        v, perm.reshape(LANES, 1),
        lax.GatherDimensionNumbers(
            offset_dims=(), collapsed_slice_dims=(0,), start_index_map=(0,)),
        slice_sizes=(1,),
        mode=lax.GatherScatterMode.PROMISE_IN_BOUNDS)


def _sc_kernel_body(x_hbm, src_hbm, dst_hbm, out_hbm,
                    idx_s, idx_d, rows_s, rows_d, scores,
                    sem_s, sem_d):
    wid = lax.axis_index("s") * NUM_CORES + lax.axis_index("c")
    worker_base = wid * EDGES_PER_WORKER
    lane_iota = lax.iota(jnp.int32, LANES)
    zeros_f32 = (lane_iota - lane_iota).astype(jnp.float32)

    def block_body(blk, carry):
        base = worker_base + blk * BLOCK_E
        pltpu.sync_copy(src_hbm.at[pl.ds(base, BLOCK_PAD)], idx_s)
        pltpu.sync_copy(dst_hbm.at[pl.ds(base, BLOCK_PAD)], idx_d)
        cp_s = pltpu.async_copy(x_hbm.at[idx_s], rows_s, sem_s)
        cp_d = pltpu.async_copy(x_hbm.at[idx_d], rows_d, sem_d)
        cp_s.wait()
        cp_d.wait()

        def group_body(g, carry2):
            row0 = g * LANES
            res = zeros_f32
            for e in range(LANES):
                row = row0 + e
                acc = (rows_s[row, pl.ds(0, LANES)]
                       * rows_d[row, pl.ds(0, LANES)])
                for c in range(1, D_CHUNKS):
                    acc = acc + (rows_s[row, pl.ds(c * LANES, LANES)]
                                 * rows_d[row, pl.ds(c * LANES, LANES)])
                for sh in (1, 2, 4, 8):
                    acc = acc + _permute(acc, lane_iota ^ sh)
                res = jnp.where(lane_iota == e, acc, res)
            scores[pl.ds(row0, LANES)] = res
            return carry2

        lax.fori_loop(0, NUM_GROUPS, group_body, 0)
        pltpu.sync_copy(scores.at[pl.ds(0, BLOCK_E)],
                        out_hbm.at[pl.ds(base, BLOCK_E)])
        return carry

    lax.fori_loop(0, NUM_BLOCKS, block_body, 0)


def kernel(x, edge_index):
    pad = jnp.zeros((BLOCK_PAD - BLOCK_E,), jnp.int32)
    src = jnp.concatenate([edge_index[0], pad])
    dst = jnp.concatenate([edge_index[1], pad])

    mesh = plsc.VectorSubcoreMesh(core_axis_name="c", subcore_axis_name="s")
    run = pl.kernel(
        _sc_kernel_body,
        mesh=mesh,
        out_type=jax.ShapeDtypeStruct((N_EDGES,), jnp.float32),
        scratch_types=[
            pltpu.VMEM((BLOCK_PAD,), jnp.int32),
            pltpu.VMEM((BLOCK_PAD,), jnp.int32),
            pltpu.VMEM((BLOCK_PAD, D_FEAT), jnp.float32),
            pltpu.VMEM((BLOCK_PAD, D_FEAT), jnp.float32),
            pltpu.VMEM((BLOCK_PAD,), jnp.float32),
            pltpu.SemaphoreType.DMA,
            pltpu.SemaphoreType.DMA,
        ],
    )
    score = run(x, src, dst)
    return score.reshape(N_EDGES, 1)


# speedup vs baseline: 3.6218x; 2.4635x over previous
import jax
import jax.numpy as jnp
from jax import lax
from jax.experimental import pallas as pl
from jax.experimental.pallas import tpu as pltpu
from jax.experimental.pallas import tpu_sc as plsc

N_NODES = 10000
N_EDGES = 160000
D_FEAT = 256

NUM_CORES = 2
NUM_SUBCORES = 16
NUM_WORKERS = NUM_CORES * NUM_SUBCORES

EDGES_PER_WORKER = N_EDGES // NUM_WORKERS
BLOCK_E = 200
LANES = 16
NUM_GROUPS = (BLOCK_E + LANES - 1) // LANES
BLOCK_PAD = NUM_GROUPS * LANES
NUM_BLOCKS = EDGES_PER_WORKER // BLOCK_E
D_WORDS = D_FEAT // 2
D_PAIRS = D_WORDS // LANES


def _permute(v, perm):
    return lax.gather(
        v, perm.reshape(LANES, 1),
        lax.GatherDimensionNumbers(
            offset_dims=(), collapsed_slice_dims=(0,), start_index_map=(0,)),
        slice_sizes=(1,),
        mode=lax.GatherScatterMode.PROMISE_IN_BOUNDS)


def _sc_kernel_body(x_hbm, src_hbm, dst_hbm, out_hbm,
                    idx_s0, idx_d0, idx_s1, idx_d1,
                    rows_s0, rows_d0, rows_s1, rows_d1, scores,
                    sem_s0, sem_d0, sem_s1, sem_d1):
    wid = lax.axis_index("s") * NUM_CORES + lax.axis_index("c")
    worker_base = wid * EDGES_PER_WORKER
    lane_iota = lax.iota(jnp.int32, LANES)
    zeros_f32 = (lane_iota - lane_iota).astype(jnp.float32)

    idx_bufs = ((idx_s0, idx_d0), (idx_s1, idx_d1))
    row_bufs = ((rows_s0, rows_d0), (rows_s1, rows_d1))
    sems = ((sem_s0, sem_d0), (sem_s1, sem_d1))

    def issue(blk, slot):
        base = worker_base + blk * BLOCK_E
        idx_s, idx_d = idx_bufs[slot]
        rows_s, rows_d = row_bufs[slot]
        sem_s, sem_d = sems[slot]
        pltpu.sync_copy(src_hbm.at[pl.ds(base, BLOCK_PAD)], idx_s)
        pltpu.sync_copy(dst_hbm.at[pl.ds(base, BLOCK_PAD)], idx_d)
        pltpu.async_copy(x_hbm.at[idx_s], rows_s, sem_s)
        pltpu.async_copy(x_hbm.at[idx_d], rows_d, sem_d)

    def wait(slot):
        idx_s, idx_d = idx_bufs[slot]
        rows_s, rows_d = row_bufs[slot]
        sem_s, sem_d = sems[slot]
        pltpu.make_async_copy(x_hbm.at[idx_s], rows_s, sem_s).wait()
        pltpu.make_async_copy(x_hbm.at[idx_d], rows_d, sem_d).wait()

    def compute_and_flush(blk, slot):
        base = worker_base + blk * BLOCK_E
        rows_s, rows_d = row_bufs[slot]

        def group_body(g, carry):
            row0 = g * LANES
            res = zeros_f32
            for e in range(LANES):
                row = row0 + e
                acc = zeros_f32
                for p in range(D_PAIRS):
                    ws = rows_s[row, pl.ds(p * LANES, LANES)]
                    wd = rows_d[row, pl.ds(p * LANES, LANES)]
                    s_lo = lax.bitcast_convert_type(ws << 16, jnp.float32)
                    d_lo = lax.bitcast_convert_type(wd << 16, jnp.float32)
                    s_hi = lax.bitcast_convert_type(
                        ws & jnp.int32(-65536), jnp.float32)
                    d_hi = lax.bitcast_convert_type(
                        wd & jnp.int32(-65536), jnp.float32)
                    acc = acc + s_lo * d_lo + s_hi * d_hi
                for sh in (1, 2, 4, 8):
                    acc = acc + _permute(acc, lane_iota ^ sh)
                res = jnp.where(lane_iota == e, acc, res)
            scores[pl.ds(row0, LANES)] = res
            return carry

        lax.fori_loop(0, NUM_GROUPS, group_body, 0)
        pltpu.sync_copy(scores.at[pl.ds(0, BLOCK_E)],
                        out_hbm.at[pl.ds(base, BLOCK_E)])

    issue(0, 0)

    def pair_body(i, carry):
        blk0 = i * 2
        wait(0)
        issue(blk0 + 1, 1)
        compute_and_flush(blk0, 0)
        wait(1)
        issue(blk0 + 2, 0)
        compute_and_flush(blk0 + 1, 1)
        return carry

    lax.fori_loop(0, (NUM_BLOCKS - 1) // 2, pair_body, 0)
    wait(0)
    compute_and_flush(NUM_BLOCKS - 1, 0)


def kernel(x, edge_index):
    xw = lax.bitcast_convert_type(
        x.astype(jnp.bfloat16).reshape(N_NODES, D_WORDS, 2), jnp.int32)
    pad = jnp.zeros((BLOCK_PAD - BLOCK_E,), jnp.int32)
    src = jnp.concatenate([edge_index[0], pad])
    dst = jnp.concatenate([edge_index[1], pad])

    mesh = plsc.VectorSubcoreMesh(core_axis_name="c", subcore_axis_name="s")
    run = pl.kernel(
        _sc_kernel_body,
        mesh=mesh,
        out_type=jax.ShapeDtypeStruct((N_EDGES,), jnp.float32),
        scratch_types=[
            pltpu.VMEM((BLOCK_PAD,), jnp.int32),
            pltpu.VMEM((BLOCK_PAD,), jnp.int32),
            pltpu.VMEM((BLOCK_PAD,), jnp.int32),
            pltpu.VMEM((BLOCK_PAD,), jnp.int32),
            pltpu.VMEM((BLOCK_PAD, D_WORDS), jnp.int32),
            pltpu.VMEM((BLOCK_PAD, D_WORDS), jnp.int32),
            pltpu.VMEM((BLOCK_PAD, D_WORDS), jnp.int32),
            pltpu.VMEM((BLOCK_PAD, D_WORDS), jnp.int32),
            pltpu.VMEM((BLOCK_PAD,), jnp.float32),
            pltpu.SemaphoreType.DMA,
            pltpu.SemaphoreType.DMA,
            pltpu.SemaphoreType.DMA,
            pltpu.SemaphoreType.DMA,
        ],
    )
    score = run(xw, src, dst)
    return score.reshape(N_EDGES, 1)
